# baseline (speedup 1.0000x reference)
import functools
import math

import jax
import jax.numpy as jnp
from jax import lax
from jax.experimental import pallas as pl
from jax.experimental.pallas import tpu as pltpu
from jax.experimental.pallas import tpu_sc as plsc

D_MODEL = 64
SEQ_LEN = 200
BATCH = 4096
MAX_LEN = 350

_NC = 2
_NS = 16
_NW = _NC * _NS
_B_TOTAL = BATCH * SEQ_LEN
_B_PER_W = _B_TOTAL // _NW
_CHUNK = 800
_N_CHUNKS = _B_PER_W // _CHUNK
_GROUPS = D_MODEL // 16


def _sin_pos_encoding():
    position = jnp.arange(0, MAX_LEN, dtype=jnp.float32)[:, None]
    divisor = jnp.exp(
        jnp.arange(0, D_MODEL, 2, dtype=jnp.float32)
        * -(math.log(10000.0) / D_MODEL)
    )
    pe = jnp.zeros((MAX_LEN, D_MODEL), dtype=jnp.float32)
    pe = pe.at[:, 0::2].set(jnp.sin(position * divisor))
    pe = pe.at[:, 1::2].set(jnp.cos(position * divisor))
    return pe[:SEQ_LEN]


@functools.partial(
    pl.kernel,
    out_type=jax.ShapeDtypeStruct((_B_TOTAL, D_MODEL), jnp.float32),
    mesh=plsc.VectorSubcoreMesh(core_axis_name="c", subcore_axis_name="s"),
    scratch_types=[
        pltpu.VMEM((_B_PER_W,), jnp.int32),
        pltpu.VMEM((_CHUNK, D_MODEL), jnp.float32),
        pltpu.VMEM((SEQ_LEN, D_MODEL), jnp.float32),
        pltpu.SemaphoreType.DMA,
    ],
    compiler_params=pltpu.CompilerParams(use_tc_tiling_on_sc=False),
)
def _sc_embed(table_hbm, x_hbm, pe_hbm, out_hbm, idx_v, rows_v, pe_v, sem):
    wid = lax.axis_index("s") * _NC + lax.axis_index("c")
    base = wid * _B_PER_W
    scale = float(math.sqrt(D_MODEL))

    pltpu.sync_copy(pe_hbm, pe_v)
    pltpu.sync_copy(x_hbm.at[pl.ds(base, _B_PER_W)], idx_v)

    def chunk_body(t, carry):
        for u in range(2):
            g = t * 2 + u
            off = base + g * _CHUNK
            pltpu.async_copy(
                table_hbm.at[idx_v.at[pl.ds(g * _CHUNK, _CHUNK)]], rows_v, sem
            ).wait()

            def pos_body(s, c2):
                pev = [pe_v[s, pl.ds(gi * 16, 16)] for gi in range(_GROUPS)]
                for m in range(_CHUNK // SEQ_LEN):
                    r = m * SEQ_LEN + s
                    for gi in range(_GROUPS):
                        sl = pl.ds(gi * 16, 16)
                        rows_v[r, sl] = rows_v[r, sl] * scale + pev[gi]
                return c2

            lax.fori_loop(0, SEQ_LEN, pos_body, 0)
            pltpu.sync_copy(rows_v, out_hbm.at[pl.ds(off, _CHUNK)])
        return carry

    lax.fori_loop(0, _N_CHUNKS // 2, chunk_body, 0)


def kernel(x, emb_table):
    pe = _sin_pos_encoding()
    out = _sc_embed(emb_table, x.reshape(-1), pe)
    return out.reshape(BATCH, SEQ_LEN, D_MODEL)


# speedup vs baseline: 1.0594x; 1.0594x over previous
import functools
import math

import jax
import jax.numpy as jnp
from jax import lax
from jax.experimental import pallas as pl
from jax.experimental.pallas import tpu as pltpu
from jax.experimental.pallas import tpu_sc as plsc

D_MODEL = 64
SEQ_LEN = 200
BATCH = 4096
MAX_LEN = 350

_NC = 2
_NS = 16
_NW = _NC * _NS
_B_TOTAL = BATCH * SEQ_LEN
_B_PER_W = _B_TOTAL // _NW
_CHUNK = 400
_N_CHUNKS = _B_PER_W // _CHUNK
_GROUPS = D_MODEL // 16


def _sin_pos_encoding():
    position = jnp.arange(0, MAX_LEN, dtype=jnp.float32)[:, None]
    divisor = jnp.exp(
        jnp.arange(0, D_MODEL, 2, dtype=jnp.float32)
        * -(math.log(10000.0) / D_MODEL)
    )
    pe = jnp.zeros((MAX_LEN, D_MODEL), dtype=jnp.float32)
    pe = pe.at[:, 0::2].set(jnp.sin(position * divisor))
    pe = pe.at[:, 1::2].set(jnp.cos(position * divisor))
    return pe[:SEQ_LEN]


@functools.partial(
    pl.kernel,
    out_type=jax.ShapeDtypeStruct((_B_TOTAL, D_MODEL), jnp.float32),
    mesh=plsc.VectorSubcoreMesh(core_axis_name="c", subcore_axis_name="s"),
    scratch_types=[
        pltpu.VMEM((_B_PER_W,), jnp.int32),
        pltpu.VMEM((_CHUNK, D_MODEL), jnp.float32),
        pltpu.VMEM((_CHUNK, D_MODEL), jnp.float32),
        pltpu.VMEM((SEQ_LEN, D_MODEL), jnp.float32),
        pltpu.SemaphoreType.DMA,
        pltpu.SemaphoreType.DMA,
        pltpu.SemaphoreType.DMA,
        pltpu.SemaphoreType.DMA,
    ],
    compiler_params=pltpu.CompilerParams(use_tc_tiling_on_sc=False),
)
def _sc_embed(table_hbm, x_hbm, pe_hbm, out_hbm,
              idx_v, rows0, rows1, pe_v, sg0, sg1, sw0, sw1):
    wid = lax.axis_index("s") * _NC + lax.axis_index("c")
    base = wid * _B_PER_W
    scale = float(math.sqrt(D_MODEL))
    rows = (rows0, rows1)
    sg = (sg0, sg1)
    sw = (sw0, sw1)

    pltpu.sync_copy(pe_hbm, pe_v)
    pltpu.sync_copy(x_hbm.at[pl.ds(base, _B_PER_W)], idx_v)

    def gather_src(g):
        return table_hbm.at[idx_v.at[pl.ds(g * _CHUNK, _CHUNK)]]

    def out_dst(g):
        return out_hbm.at[pl.ds(base + g * _CHUNK, _CHUNK)]

    pltpu.async_copy(gather_src(0), rows0, sg0)

    def outer(t, carry):
        for b in range(2):
            g = t * 2 + b
            nb = 1 - b

            @pl.when(g + 1 < _N_CHUNKS)
            def _issue_next():
                @pl.when(g >= 1)
                def _drain_wb():
                    pltpu.make_async_copy(rows[nb], out_dst(g - 1), sw[nb]).wait()
                pltpu.async_copy(gather_src(g + 1), rows[nb], sg[nb])

            pltpu.make_async_copy(gather_src(g), rows[b], sg[b]).wait()

            def pos_body(s, c2):
                pev = [pe_v[s, pl.ds(gi * 16, 16)] for gi in range(_GROUPS)]
                for m in range(_CHUNK // SEQ_LEN):
                    r = m * SEQ_LEN + s
                    for gi in range(_GROUPS):
                        sl = pl.ds(gi * 16, 16)
                        rows[b][r, sl] = rows[b][r, sl] * scale + pev[gi]
                return c2

            lax.fori_loop(0, SEQ_LEN, pos_body, 0)
            pltpu.async_copy(rows[b], out_dst(g), sw[b])
        return carry

    lax.fori_loop(0, _N_CHUNKS // 2, outer, 0)
    pltpu.make_async_copy(rows[0], out_dst(_N_CHUNKS - 2), sw[0]).wait()
    pltpu.make_async_copy(rows[1], out_dst(_N_CHUNKS - 1), sw[1]).wait()


def kernel(x, emb_table):
    pe = _sin_pos_encoding()
    out = _sc_embed(emb_table, x.reshape(-1), pe)
    return out.reshape(BATCH, SEQ_LEN, D_MODEL)
